# Initial kernel scaffold; baseline (speedup 1.0000x reference)
import jax
import jax.numpy as jnp
from jax.experimental import pallas as pl


def kernel(x, edge_index, Wl1, bl1, Wr1, Wl2, bl2, Wr2, Wl3, bl3, Wr3, Wreg, breg):
    raise NotImplementedError("write your pallas kernel here")


# speedup vs baseline: 4.8030x; 4.8030x over previous
import functools

import jax
import jax.numpy as jnp
from jax import lax
from jax.experimental import pallas as pl
from jax.experimental.pallas import tpu as pltpu
from jax.experimental.pallas import tpu_sc as plsc

NC = 2
NS = 16
CHUNK = 80


def _slab(n):
  return (-(-n // NS) + 7) // 8 * 8


def _sc_aggregate(h, src, dst):
  N, D = h.shape
  E = src.shape[0]
  n_workers = NC * NS
  iters = E // (CHUNK * n_workers)
  slab = _slab(N)
  npad = slab * NS
  mesh = plsc.VectorSubcoreMesh(core_axis_name="c", subcore_axis_name="s")

  @functools.partial(
      pl.kernel, mesh=mesh,
      out_type=jax.ShapeDtypeStruct((NC, npad, D), jnp.float32),
      scratch_types=[
          pltpu.VMEM_SHARED((npad, D), jnp.float32),
          pltpu.VMEM((CHUNK,), jnp.int32),
          pltpu.VMEM((CHUNK,), jnp.int32),
          pltpu.VMEM((CHUNK, D), jnp.float32),
          pltpu.SemaphoreType.DMA,
      ])
  def k(h_hbm, src_hbm, dst_hbm, z_hbm, out_hbm, acc_sp, src_v, dst_v,
        rows_v, sem):
    cid = lax.axis_index("c")
    sid = lax.axis_index("s")
    wid = sid * NC + cid
    row0 = sid * slab
    pltpu.sync_copy(z_hbm, acc_sp.at[pl.ds(row0, slab)])
    plsc.subcore_barrier()

    def step(t, carry):
      e0 = (wid + n_workers * t) * CHUNK
      pltpu.sync_copy(src_hbm.at[pl.ds(e0, CHUNK)], src_v)
      pltpu.sync_copy(dst_hbm.at[pl.ds(e0, CHUNK)], dst_v)
      pltpu.async_copy(h_hbm.at[src_v], rows_v, sem).wait()
      pltpu.sync_copy(rows_v, acc_sp.at[dst_v], add=True)
      return carry
    lax.fori_loop(0, iters, step, 0)
    plsc.subcore_barrier()

    pltpu.sync_copy(acc_sp.at[pl.ds(row0, slab)],
                    out_hbm.at[cid, pl.ds(row0, slab)])

  z = jnp.zeros((slab, D), jnp.float32)
  return k(h, src, dst, z)


def _sc_degree(dst, N, D):
  E = dst.shape[0]
  n_workers = NC * NS
  iters = E // (CHUNK * n_workers)
  slab = _slab(N)
  npad = slab * NS
  mesh = plsc.VectorSubcoreMesh(core_axis_name="c", subcore_axis_name="s")

  @functools.partial(
      pl.kernel, mesh=mesh,
      out_type=jax.ShapeDtypeStruct((NC, npad, D), jnp.float32),
      scratch_types=[
          pltpu.VMEM_SHARED((npad, D), jnp.float32),
          pltpu.VMEM((CHUNK,), jnp.int32),
          pltpu.VMEM((CHUNK, D), jnp.float32),
      ])
  def k(dst_hbm, z_hbm, ones_hbm, out_hbm, acc_sp, dst_v, ones_v):
    cid = lax.axis_index("c")
    sid = lax.axis_index("s")
    wid = sid * NC + cid
    row0 = sid * slab
    pltpu.sync_copy(z_hbm, acc_sp.at[pl.ds(row0, slab)])
    pltpu.sync_copy(ones_hbm, ones_v)
    plsc.subcore_barrier()

    def step(t, carry):
      e0 = (wid + n_workers * t) * CHUNK
      pltpu.sync_copy(dst_hbm.at[pl.ds(e0, CHUNK)], dst_v)
      pltpu.sync_copy(ones_v, acc_sp.at[dst_v], add=True)
      return carry
    lax.fori_loop(0, iters, step, 0)
    plsc.subcore_barrier()

    pltpu.sync_copy(acc_sp.at[pl.ds(row0, slab)],
                    out_hbm.at[cid, pl.ds(row0, slab)])

  z = jnp.zeros((slab, D), jnp.float32)
  ones = jnp.ones((CHUNK, D), jnp.float32)
  return k(dst, z, ones)


def _tc_layer(parts, degp, h, Wl, bl, Wr, relu, Wreg=None, breg=None):
  N, D = h.shape
  BN = 2000
  final = Wreg is not None

  def body(*refs):
    if final:
      p_ref, d_ref, h_ref, wl_ref, b_ref, wr_ref, wg_ref, bg_ref, o_ref = refs
    else:
      p_ref, d_ref, h_ref, wl_ref, b_ref, wr_ref, o_ref = refs
    agg = p_ref[0] + p_ref[1]
    deg = d_ref[0, :, 0:1] + d_ref[1, :, 0:1]
    inv = 1.0 / jnp.maximum(deg, 1.0)
    y = jnp.dot(agg * inv, wl_ref[...].T, preferred_element_type=jnp.float32)
    y += jnp.dot(h_ref[...], wr_ref[...].T, preferred_element_type=jnp.float32)
    y += b_ref[...]
    if relu:
      y = jnp.maximum(y, 0.0)
    if final:
      y = jnp.dot(y, wg_ref[...].T, preferred_element_type=jnp.float32)
      y += bg_ref[...]
    o_ref[...] = y

  in_specs = [
      pl.BlockSpec((NC, BN, D), lambda i: (0, i, 0)),
      pl.BlockSpec((NC, BN, D), lambda i: (0, i, 0)),
      pl.BlockSpec((BN, D), lambda i: (i, 0)),
      pl.BlockSpec((D, D), lambda i: (0, 0)),
      pl.BlockSpec((1, D), lambda i: (0, 0)),
      pl.BlockSpec((D, D), lambda i: (0, 0)),
  ]
  args = [parts, degp, h, Wl, bl.reshape(1, D), Wr]
  if final:
    in_specs += [pl.BlockSpec((D, D), lambda i: (0, 0)),
                 pl.BlockSpec((1, D), lambda i: (0, 0))]
    args += [Wreg, breg.reshape(1, D)]

  return pl.pallas_call(
      body,
      grid=(N // BN,),
      in_specs=in_specs,
      out_specs=pl.BlockSpec((BN, D), lambda i: (i, 0)),
      out_shape=jax.ShapeDtypeStruct((N, D), jnp.float32),
  )(*args)


def kernel(x, edge_index, Wl1, bl1, Wr1, Wl2, bl2, Wr2, Wl3, bl3, Wr3,
           Wreg, breg):
  src = edge_index[0]
  dst = edge_index[1]
  N, D = x.shape
  degp = _sc_degree(dst, N, D)
  p1 = _sc_aggregate(x, src, dst)
  h1 = _tc_layer(p1, degp, x, Wl1, bl1, Wr1, relu=True)
  p2 = _sc_aggregate(h1, src, dst)
  h2 = _tc_layer(p2, degp, h1, Wl2, bl2, Wr2, relu=True)
  p3 = _sc_aggregate(h2, src, dst)
  return _tc_layer(p3, degp, h2, Wl3, bl3, Wr3, relu=False,
                   Wreg=Wreg, breg=breg)


# speedup vs baseline: 9.2871x; 1.9336x over previous
import functools

import jax
import jax.numpy as jnp
from jax import lax
from jax.experimental import pallas as pl
from jax.experimental.pallas import tpu as pltpu
from jax.experimental.pallas import tpu_sc as plsc

NC = 2
NS = 16
CHUNK = 80


def _slab(n):
  return (-(-n // NS) + 7) // 8 * 8


def _sc_aggregate(h, src, dst):
  N, D = h.shape
  E = src.shape[0]
  n_workers = NC * NS
  iters = E // (CHUNK * n_workers)
  slab = _slab(N)
  npad = slab * NS
  mesh = plsc.VectorSubcoreMesh(core_axis_name="c", subcore_axis_name="s")

  @functools.partial(
      pl.kernel, mesh=mesh,
      out_type=jax.ShapeDtypeStruct((NC, npad, D), jnp.float32),
      scratch_types=[
          pltpu.VMEM_SHARED((npad, D), jnp.float32),
          pltpu.VMEM((CHUNK,), jnp.int32),
          pltpu.VMEM((CHUNK,), jnp.int32),
          pltpu.VMEM((CHUNK,), jnp.int32),
          pltpu.VMEM((CHUNK,), jnp.int32),
          pltpu.VMEM((CHUNK, D), jnp.float32),
          pltpu.VMEM((CHUNK, D), jnp.float32),
          pltpu.SemaphoreType.DMA,
          pltpu.SemaphoreType.DMA,
          pltpu.SemaphoreType.DMA,
          pltpu.SemaphoreType.DMA,
      ])
  def k(h_hbm, src_hbm, dst_hbm, z_hbm, out_hbm, acc_sp, src_v0, src_v1,
        dst_v0, dst_v1, rows_v0, rows_v1, sem_i0, sem_i1, sem_g0, sem_g1):
    cid = lax.axis_index("c")
    sid = lax.axis_index("s")
    wid = sid * NC + cid
    row0 = sid * slab
    src_vs, dst_vs = (src_v0, src_v1), (dst_v0, dst_v1)
    rows_vs = (rows_v0, rows_v1)
    sem_is, sem_gs = (sem_i0, sem_i1), (sem_g0, sem_g1)

    def e0_of(t):
      return (wid + n_workers * t) * CHUNK

    def idx_start(t, b):
      e0 = e0_of(jnp.minimum(t, iters - 1))
      pltpu.make_async_copy(src_hbm.at[pl.ds(e0, CHUNK)], src_vs[b],
                            sem_is[b]).start()
      pltpu.make_async_copy(dst_hbm.at[pl.ds(e0, CHUNK)], dst_vs[b],
                            sem_is[b]).start()

    def idx_wait(b):
      pltpu.make_async_copy(src_hbm.at[pl.ds(0, CHUNK)], src_vs[b],
                            sem_is[b]).wait()
      pltpu.make_async_copy(dst_hbm.at[pl.ds(0, CHUNK)], dst_vs[b],
                            sem_is[b]).wait()

    def gather_start(b):
      pltpu.make_async_copy(h_hbm.at[src_vs[b]], rows_vs[b],
                            sem_gs[b]).start()

    def gather_wait(b):
      pltpu.make_async_copy(h_hbm.at[src_vs[b]], rows_vs[b],
                            sem_gs[b]).wait()

    pltpu.sync_copy(z_hbm, acc_sp.at[pl.ds(row0, slab)])
    plsc.subcore_barrier()

    pltpu.sync_copy(src_hbm.at[pl.ds(e0_of(0), CHUNK)], src_v0)
    pltpu.sync_copy(dst_hbm.at[pl.ds(e0_of(0), CHUNK)], dst_v0)
    idx_start(1, 1)
    gather_start(0)

    def pair(tt, carry):
      for b in range(2):
        t = 2 * tt + b
        nb = 1 - b
        idx_wait(nb)
        gather_start(nb)
        gather_wait(b)
        pltpu.sync_copy(rows_vs[b], acc_sp.at[dst_vs[b]], add=True)
        idx_start(t + 2, b)
      return carry
    lax.fori_loop(0, (iters - 1) // 2, pair, 0)

    gather_wait(0)
    pltpu.sync_copy(rows_v0, acc_sp.at[dst_v0], add=True)
    idx_wait(1)
    plsc.subcore_barrier()

    pltpu.sync_copy(acc_sp.at[pl.ds(row0, slab)],
                    out_hbm.at[cid, pl.ds(row0, slab)])

  z = jnp.zeros((slab, D), jnp.float32)
  return k(h, src, dst, z)


def _sc_degree(dst, N, D):
  E = dst.shape[0]
  n_workers = NC * NS
  iters = E // (CHUNK * n_workers)
  slab = _slab(N)
  npad = slab * NS
  mesh = plsc.VectorSubcoreMesh(core_axis_name="c", subcore_axis_name="s")

  @functools.partial(
      pl.kernel, mesh=mesh,
      out_type=jax.ShapeDtypeStruct((NC, npad, D), jnp.float32),
      scratch_types=[
          pltpu.VMEM_SHARED((npad, D), jnp.float32),
          pltpu.VMEM((CHUNK,), jnp.int32),
          pltpu.VMEM((CHUNK,), jnp.int32),
          pltpu.VMEM((CHUNK, D), jnp.float32),
          pltpu.SemaphoreType.DMA,
          pltpu.SemaphoreType.DMA,
      ])
  def k(dst_hbm, z_hbm, ones_hbm, out_hbm, acc_sp, dst_v0, dst_v1, ones_v,
        sem_i0, sem_i1):
    cid = lax.axis_index("c")
    sid = lax.axis_index("s")
    wid = sid * NC + cid
    row0 = sid * slab
    dst_vs = (dst_v0, dst_v1)
    sem_is = (sem_i0, sem_i1)

    def e0_of(t):
      return (wid + n_workers * t) * CHUNK

    def idx_start(t, b):
      e0 = e0_of(jnp.minimum(t, iters - 1))
      pltpu.make_async_copy(dst_hbm.at[pl.ds(e0, CHUNK)], dst_vs[b],
                            sem_is[b]).start()

    def idx_wait(b):
      pltpu.make_async_copy(dst_hbm.at[pl.ds(0, CHUNK)], dst_vs[b],
                            sem_is[b]).wait()

    pltpu.sync_copy(z_hbm, acc_sp.at[pl.ds(row0, slab)])
    pltpu.sync_copy(ones_hbm, ones_v)
    plsc.subcore_barrier()

    pltpu.sync_copy(dst_hbm.at[pl.ds(e0_of(0), CHUNK)], dst_v0)
    idx_start(1, 1)

    def pair(tt, carry):
      for b in range(2):
        t = 2 * tt + b
        nb = 1 - b
        pltpu.sync_copy(ones_v, acc_sp.at[dst_vs[b]], add=True)
        idx_start(t + 2, b)
        idx_wait(nb)
      return carry
    lax.fori_loop(0, (iters - 1) // 2, pair, 0)

    pltpu.sync_copy(ones_v, acc_sp.at[dst_v0], add=True)
    idx_wait(1)
    plsc.subcore_barrier()

    pltpu.sync_copy(acc_sp.at[pl.ds(row0, slab)],
                    out_hbm.at[cid, pl.ds(row0, slab)])

  z = jnp.zeros((slab, D), jnp.float32)
  ones = jnp.ones((CHUNK, D), jnp.float32)
  return k(dst, z, ones)


def _tc_layer(parts, degp, h, Wl, bl, Wr, relu, Wreg=None, breg=None):
  N, D = h.shape
  BN = 2000
  final = Wreg is not None

  def body(*refs):
    if final:
      p_ref, d_ref, h_ref, wl_ref, b_ref, wr_ref, wg_ref, bg_ref, o_ref = refs
    else:
      p_ref, d_ref, h_ref, wl_ref, b_ref, wr_ref, o_ref = refs
    agg = p_ref[0] + p_ref[1]
    deg = d_ref[0, :, 0:1] + d_ref[1, :, 0:1]
    inv = 1.0 / jnp.maximum(deg, 1.0)
    y = jnp.dot(agg * inv, wl_ref[...].T, preferred_element_type=jnp.float32)
    y += jnp.dot(h_ref[...], wr_ref[...].T, preferred_element_type=jnp.float32)
    y += b_ref[...]
    if relu:
      y = jnp.maximum(y, 0.0)
    if final:
      y = jnp.dot(y, wg_ref[...].T, preferred_element_type=jnp.float32)
      y += bg_ref[...]
    o_ref[...] = y

  in_specs = [
      pl.BlockSpec((NC, BN, D), lambda i: (0, i, 0)),
      pl.BlockSpec((NC, BN, D), lambda i: (0, i, 0)),
      pl.BlockSpec((BN, D), lambda i: (i, 0)),
      pl.BlockSpec((D, D), lambda i: (0, 0)),
      pl.BlockSpec((1, D), lambda i: (0, 0)),
      pl.BlockSpec((D, D), lambda i: (0, 0)),
  ]
  args = [parts, degp, h, Wl, bl.reshape(1, D), Wr]
  if final:
    in_specs += [pl.BlockSpec((D, D), lambda i: (0, 0)),
                 pl.BlockSpec((1, D), lambda i: (0, 0))]
    args += [Wreg, breg.reshape(1, D)]

  return pl.pallas_call(
      body,
      grid=(N // BN,),
      in_specs=in_specs,
      out_specs=pl.BlockSpec((BN, D), lambda i: (i, 0)),
      out_shape=jax.ShapeDtypeStruct((N, D), jnp.float32),
  )(*args)


def kernel(x, edge_index, Wl1, bl1, Wr1, Wl2, bl2, Wr2, Wl3, bl3, Wr3,
           Wreg, breg):
  src = edge_index[0]
  dst = edge_index[1]
  N, D = x.shape
  degp = _sc_degree(dst, N, D)
  p1 = _sc_aggregate(x, src, dst)
  h1 = _tc_layer(p1, degp, x, Wl1, bl1, Wr1, relu=True)
  p2 = _sc_aggregate(h1, src, dst)
  h2 = _tc_layer(p2, degp, h1, Wl2, bl2, Wr2, relu=True)
  p3 = _sc_aggregate(h2, src, dst)
  return _tc_layer(p3, degp, h2, Wl3, bl3, Wr3, relu=False,
                   Wreg=Wreg, breg=breg)


# speedup vs baseline: 10.8425x; 1.1675x over previous
import functools

import jax
import jax.numpy as jnp
from jax import lax
from jax.experimental import pallas as pl
from jax.experimental.pallas import tpu as pltpu
from jax.experimental.pallas import tpu_sc as plsc

NC = 2
NS = 16
CHUNK = 80


def _slab(n):
  return (-(-n // NS) + 7) // 8 * 8


def _sc_aggregate(h, src, dst):
  N, D = h.shape
  E = src.shape[0]
  n_workers = NC * NS
  iters = E // (CHUNK * n_workers)
  slab = _slab(N)
  npad = slab * NS
  NB = 3
  mesh = plsc.VectorSubcoreMesh(core_axis_name="c", subcore_axis_name="s")

  @functools.partial(
      pl.kernel, mesh=mesh,
      out_type=jax.ShapeDtypeStruct((NC, npad, D), jnp.float32),
      scratch_types=(
          [pltpu.VMEM_SHARED((npad, D), jnp.float32)]
          + [pltpu.VMEM((CHUNK,), jnp.int32)] * NB
          + [pltpu.VMEM((CHUNK,), jnp.int32)] * NB
          + [pltpu.VMEM((CHUNK, D), jnp.float32)] * NB
          + [pltpu.SemaphoreType.DMA] * NB
          + [pltpu.SemaphoreType.DMA] * NB
      ))
  def k(h_hbm, src_hbm, dst_hbm, z_hbm, out_hbm, acc_sp, *bufs):
    src_vs = bufs[0:NB]
    dst_vs = bufs[NB:2 * NB]
    rows_vs = bufs[2 * NB:3 * NB]
    sem_is = bufs[3 * NB:4 * NB]
    sem_gs = bufs[4 * NB:5 * NB]
    cid = lax.axis_index("c")
    sid = lax.axis_index("s")
    wid = sid * NC + cid
    row0 = sid * slab

    def e0_of(t):
      return (wid + n_workers * t) * CHUNK

    def idx_start(t, b):
      e0 = e0_of(jnp.minimum(t, iters - 1))
      pltpu.make_async_copy(src_hbm.at[pl.ds(e0, CHUNK)], src_vs[b],
                            sem_is[b]).start()
      pltpu.make_async_copy(dst_hbm.at[pl.ds(e0, CHUNK)], dst_vs[b],
                            sem_is[b]).start()

    def idx_wait(b):
      pltpu.make_async_copy(src_hbm.at[pl.ds(0, CHUNK)], src_vs[b],
                            sem_is[b]).wait()
      pltpu.make_async_copy(dst_hbm.at[pl.ds(0, CHUNK)], dst_vs[b],
                            sem_is[b]).wait()

    def gather_start(b):
      pltpu.make_async_copy(h_hbm.at[src_vs[b]], rows_vs[b],
                            sem_gs[b]).start()

    def gather_wait(b):
      pltpu.make_async_copy(h_hbm.at[src_vs[b]], rows_vs[b],
                            sem_gs[b]).wait()

    def scatter(b):
      pltpu.sync_copy(rows_vs[b], acc_sp.at[dst_vs[b]], add=True)

    pltpu.sync_copy(z_hbm, acc_sp.at[pl.ds(row0, slab)])
    plsc.subcore_barrier()

    for t0 in range(2):
      pltpu.sync_copy(src_hbm.at[pl.ds(e0_of(t0), CHUNK)], src_vs[t0])
      pltpu.sync_copy(dst_hbm.at[pl.ds(e0_of(t0), CHUNK)], dst_vs[t0])
      gather_start(t0)
    idx_start(2, 2)

    def triple(tt, carry):
      for b in range(NB):
        t = NB * tt + b
        pb = (b + 2) % NB
        gather_wait(b)
        scatter(b)
        idx_wait(pb)
        gather_start(pb)
        idx_start(t + NB, b)
      return carry
    lax.fori_loop(0, (iters - 2) // NB, triple, 0)

    gather_wait(0)
    scatter(0)
    gather_wait(1)
    scatter(1)
    idx_wait(2)
    plsc.subcore_barrier()

    pltpu.sync_copy(acc_sp.at[pl.ds(row0, slab)],
                    out_hbm.at[cid, pl.ds(row0, slab)])

  z = jnp.zeros((slab, D), jnp.float32)
  return k(h, src, dst, z)


def _sc_degree(dst, N, D):
  E = dst.shape[0]
  n_workers = NC * NS
  iters = E // (CHUNK * n_workers)
  slab = _slab(N)
  npad = slab * NS
  mesh = plsc.VectorSubcoreMesh(core_axis_name="c", subcore_axis_name="s")

  @functools.partial(
      pl.kernel, mesh=mesh,
      out_type=jax.ShapeDtypeStruct((NC, npad, D), jnp.float32),
      scratch_types=[
          pltpu.VMEM_SHARED((npad, D), jnp.float32),
          pltpu.VMEM((CHUNK,), jnp.int32),
          pltpu.VMEM((CHUNK,), jnp.int32),
          pltpu.VMEM((CHUNK, D), jnp.float32),
          pltpu.SemaphoreType.DMA,
          pltpu.SemaphoreType.DMA,
      ])
  def k(dst_hbm, z_hbm, ones_hbm, out_hbm, acc_sp, dst_v0, dst_v1, ones_v,
        sem_i0, sem_i1):
    cid = lax.axis_index("c")
    sid = lax.axis_index("s")
    wid = sid * NC + cid
    row0 = sid * slab
    dst_vs = (dst_v0, dst_v1)
    sem_is = (sem_i0, sem_i1)

    def e0_of(t):
      return (wid + n_workers * t) * CHUNK

    def idx_start(t, b):
      e0 = e0_of(jnp.minimum(t, iters - 1))
      pltpu.make_async_copy(dst_hbm.at[pl.ds(e0, CHUNK)], dst_vs[b],
                            sem_is[b]).start()

    def idx_wait(b):
      pltpu.make_async_copy(dst_hbm.at[pl.ds(0, CHUNK)], dst_vs[b],
                            sem_is[b]).wait()

    pltpu.sync_copy(z_hbm, acc_sp.at[pl.ds(row0, slab)])
    pltpu.sync_copy(ones_hbm, ones_v)
    plsc.subcore_barrier()

    pltpu.sync_copy(dst_hbm.at[pl.ds(e0_of(0), CHUNK)], dst_v0)
    idx_start(1, 1)

    def pair(tt, carry):
      for b in range(2):
        t = 2 * tt + b
        nb = 1 - b
        pltpu.sync_copy(ones_v, acc_sp.at[dst_vs[b]], add=True)
        idx_start(t + 2, b)
        idx_wait(nb)
      return carry
    lax.fori_loop(0, (iters - 1) // 2, pair, 0)

    pltpu.sync_copy(ones_v, acc_sp.at[dst_v0], add=True)
    idx_wait(1)
    plsc.subcore_barrier()

    pltpu.sync_copy(acc_sp.at[pl.ds(row0, slab)],
                    out_hbm.at[cid, pl.ds(row0, slab)])

  z = jnp.zeros((slab, D), jnp.float32)
  ones = jnp.ones((CHUNK, D), jnp.float32)
  return k(dst, z, ones)


def _tc_layer(parts, degp, h, Wl, bl, Wr, relu, Wreg=None, breg=None):
  N, D = h.shape
  BN = 2000
  final = Wreg is not None

  def body(*refs):
    if final:
      p_ref, d_ref, h_ref, wl_ref, b_ref, wr_ref, wg_ref, bg_ref, o_ref = refs
    else:
      p_ref, d_ref, h_ref, wl_ref, b_ref, wr_ref, o_ref = refs
    agg = p_ref[0] + p_ref[1]
    deg = d_ref[0, :, 0:1] + d_ref[1, :, 0:1]
    inv = 1.0 / jnp.maximum(deg, 1.0)
    y = jnp.dot(agg * inv, wl_ref[...].T, preferred_element_type=jnp.float32)
    y += jnp.dot(h_ref[...], wr_ref[...].T, preferred_element_type=jnp.float32)
    y += b_ref[...]
    if relu:
      y = jnp.maximum(y, 0.0)
    if final:
      y = jnp.dot(y, wg_ref[...].T, preferred_element_type=jnp.float32)
      y += bg_ref[...]
    o_ref[...] = y

  in_specs = [
      pl.BlockSpec((NC, BN, D), lambda i: (0, i, 0)),
      pl.BlockSpec((NC, BN, D), lambda i: (0, i, 0)),
      pl.BlockSpec((BN, D), lambda i: (i, 0)),
      pl.BlockSpec((D, D), lambda i: (0, 0)),
      pl.BlockSpec((1, D), lambda i: (0, 0)),
      pl.BlockSpec((D, D), lambda i: (0, 0)),
  ]
  args = [parts, degp, h, Wl, bl.reshape(1, D), Wr]
  if final:
    in_specs += [pl.BlockSpec((D, D), lambda i: (0, 0)),
                 pl.BlockSpec((1, D), lambda i: (0, 0))]
    args += [Wreg, breg.reshape(1, D)]

  return pl.pallas_call(
      body,
      grid=(N // BN,),
      in_specs=in_specs,
      out_specs=pl.BlockSpec((BN, D), lambda i: (i, 0)),
      out_shape=jax.ShapeDtypeStruct((N, D), jnp.float32),
  )(*args)


def kernel(x, edge_index, Wl1, bl1, Wr1, Wl2, bl2, Wr2, Wl3, bl3, Wr3,
           Wreg, breg):
  src = edge_index[0]
  dst = edge_index[1]
  N, D = x.shape
  degp = _sc_degree(dst, N, D)
  p1 = _sc_aggregate(x, src, dst)
  h1 = _tc_layer(p1, degp, x, Wl1, bl1, Wr1, relu=True)
  p2 = _sc_aggregate(h1, src, dst)
  h2 = _tc_layer(p2, degp, h1, Wl2, bl2, Wr2, relu=True)
  p3 = _sc_aggregate(h2, src, dst)
  return _tc_layer(p3, degp, h2, Wl3, bl3, Wr3, relu=False,
                   Wreg=Wreg, breg=breg)
